# baseline (speedup 1.0000x reference)
import functools
import math

import jax
import jax.numpy as jnp
from jax.experimental import pallas as pl
from jax.experimental.pallas import tpu as pltpu
from jax.experimental.pallas import tpu_sc as plsc

D_MODEL = 128
SCALE = math.sqrt(float(D_MODEL))
WINDOW = 128


def _sc_embed(table, idx_flat):
    num_idx = idx_flat.shape[0]
    d = table.shape[1]
    idx2d = idx_flat.reshape(1, num_idx)
    mesh = plsc.VectorSubcoreMesh(core_axis_name="core", subcore_axis_name="subcore")

    @functools.partial(
        pl.kernel,
        out_type=jax.ShapeDtypeStruct((num_idx, d), table.dtype),
        mesh=mesh,
    )
    def gather_scale(table_hbm, idx_hbm, out_hbm):
        def body(idx_vmem, out_vmem):
            pltpu.sync_copy(table_hbm.at[idx_vmem.at[0]], out_vmem)

            @pl.loop(0, WINDOW)
            def _(r):
                for c in range(0, d, 16):
                    slc = (pl.ds(r, 1), pl.ds(c, 16))
                    out_vmem[slc] = out_vmem[slc] * SCALE

        pltpu.emit_pipeline(
            body,
            grid=(num_idx // WINDOW,),
            in_specs=[pl.BlockSpec((1, WINDOW), lambda i: (0, i))],
            out_specs=[pl.BlockSpec((WINDOW, d), lambda i: (i, 0))],
            core_axis_name=("core", "subcore"),
            dimension_semantics=(pltpu.PARALLEL,),
        )(idx_hbm, out_hbm)

    return gather_scale(table, idx2d)


def kernel(x, table):
    b, s = x.shape
    idx_flat = x.reshape(-1).astype(jnp.int32)
    out = _sc_embed(table, idx_flat)
    return out.reshape(b, s, table.shape[1])


# speedup vs baseline: 3.9838x; 3.9838x over previous
import functools
import math

import jax
import jax.numpy as jnp
from jax import lax
from jax.experimental import pallas as pl
from jax.experimental.pallas import tpu as pltpu
from jax.experimental.pallas import tpu_sc as plsc

D_MODEL = 128
SCALE = math.sqrt(float(D_MODEL))

NC, NS = 2, 16
NW = NC * NS
CHUNK = 128
NBUF = 5
LEAD = 3


def _sc_embed(table, idx3d):
    nchunk = idx3d.shape[1]
    d = table.shape[1]
    num_idx = NW * nchunk * CHUNK
    per_w = nchunk * CHUNK
    mesh = plsc.VectorSubcoreMesh(core_axis_name="core", subcore_axis_name="subcore")

    scratch = [pltpu.VMEM((nchunk, CHUNK), jnp.int32)]
    scratch += [pltpu.VMEM((CHUNK, d), jnp.float32) for _ in range(NBUF)]
    scratch += [pltpu.SemaphoreType.DMA for _ in range(2 * NBUF)]

    @functools.partial(
        pl.kernel,
        out_type=jax.ShapeDtypeStruct((num_idx, d), table.dtype),
        mesh=mesh,
        scratch_types=scratch,
    )
    def gather_scale(table_hbm, idx_hbm, out_hbm, idx_v, *rest):
        bufs = rest[:NBUF]
        sg = rest[NBUF:2 * NBUF]
        ss = rest[2 * NBUF:]
        wid = lax.axis_index("subcore") * NC + lax.axis_index("core")
        base = wid * per_w

        pltpu.sync_copy(idx_hbm.at[wid], idx_v)

        def start_gather(k, b):
            pltpu.async_copy(table_hbm.at[idx_v.at[k]], bufs[b], sg[b])

        def wait_gather(k, b):
            pltpu.make_async_copy(table_hbm.at[idx_v.at[k]], bufs[b], sg[b]).wait()

        def start_store(k, b):
            pltpu.async_copy(bufs[b], out_hbm.at[pl.ds(base + k * CHUNK, CHUNK)], ss[b])

        def wait_store(b):
            pltpu.make_async_copy(
                bufs[b], out_hbm.at[pl.ds(base, CHUNK)], ss[b]
            ).wait()

        for b in range(LEAD):
            start_gather(b, b)

        @pl.loop(0, nchunk, step=NBUF)
        def _(k0):
            for i in range(NBUF):
                k = k0 + i
                b = i
                g = k + LEAD
                gb = (b + LEAD) % NBUF

                @pl.when(g < nchunk)
                def _():
                    @pl.when(g >= NBUF)
                    def _():
                        wait_store(gb)

                    start_gather(g, gb)

                wait_gather(k, b)

                @pl.loop(0, CHUNK)
                def _(r):
                    row = bufs[b].at[r]
                    for c in range(0, d, 16):
                        row[pl.ds(c, 16)] = row[pl.ds(c, 16)] * SCALE

                start_store(k, b)

        for b in range(NBUF):
            wait_store(b)

    return gather_scale(table, idx3d)


def kernel(x, table):
    b, s = x.shape
    idx3d = x.reshape(NW, -1, CHUNK).astype(jnp.int32)
    out = _sc_embed(table, idx3d)
    return out.reshape(b, s, table.shape[1])


# speedup vs baseline: 3.9845x; 1.0002x over previous
import functools
import math

import jax
import jax.numpy as jnp
from jax import lax
from jax.experimental import pallas as pl
from jax.experimental.pallas import tpu as pltpu
from jax.experimental.pallas import tpu_sc as plsc

D_MODEL = 128
SCALE = math.sqrt(float(D_MODEL))

NC, NS = 2, 16
NW = NC * NS
CHUNK = 128
NBUF = 5
LEAD = 3


def _sc_embed(table, idx_flat):
    num_idx = idx_flat.shape[0]
    d = table.shape[1]
    per_w = num_idx // NW
    nchunk = per_w // CHUNK
    mesh = plsc.VectorSubcoreMesh(core_axis_name="core", subcore_axis_name="subcore")

    scratch = [pltpu.VMEM((per_w,), jnp.int32)]
    scratch += [pltpu.VMEM((CHUNK, d), jnp.float32) for _ in range(NBUF)]
    scratch += [pltpu.SemaphoreType.DMA for _ in range(2 * NBUF)]

    @functools.partial(
        pl.kernel,
        out_type=jax.ShapeDtypeStruct((num_idx, d), table.dtype),
        mesh=mesh,
        scratch_types=scratch,
    )
    def gather_scale(table_hbm, idx_hbm, out_hbm, idx_v, *rest):
        bufs = rest[:NBUF]
        sg = rest[NBUF:2 * NBUF]
        ss = rest[2 * NBUF:]
        wid = lax.axis_index("subcore") * NC + lax.axis_index("core")
        base = wid * per_w

        pltpu.sync_copy(idx_hbm.at[pl.ds(base, per_w)], idx_v)

        def start_gather(k, b):
            pltpu.async_copy(
                table_hbm.at[idx_v.at[pl.ds(k * CHUNK, CHUNK)]], bufs[b], sg[b]
            )

        def wait_gather(k, b):
            pltpu.make_async_copy(
                table_hbm.at[idx_v.at[pl.ds(k * CHUNK, CHUNK)]], bufs[b], sg[b]
            ).wait()

        def start_store(k, b):
            pltpu.async_copy(bufs[b], out_hbm.at[pl.ds(base + k * CHUNK, CHUNK)], ss[b])

        def wait_store(b):
            pltpu.make_async_copy(
                bufs[b], out_hbm.at[pl.ds(base, CHUNK)], ss[b]
            ).wait()

        for b in range(LEAD):
            start_gather(b, b)

        @pl.loop(0, nchunk, step=NBUF)
        def _(k0):
            for i in range(NBUF):
                k = k0 + i
                b = i
                g = k + LEAD
                gb = (b + LEAD) % NBUF

                @pl.when(g < nchunk)
                def _():
                    @pl.when(g >= NBUF)
                    def _():
                        wait_store(gb)

                    start_gather(g, gb)

                wait_gather(k, b)

                @pl.loop(0, CHUNK)
                def _(r):
                    row = bufs[b].at[r]
                    for c in range(0, d, 16):
                        row[pl.ds(c, 16)] = row[pl.ds(c, 16)] * SCALE

                start_store(k, b)

        for b in range(NBUF):
            wait_store(b)

    return gather_scale(table, idx_flat)


def kernel(x, table):
    b, s = x.shape
    idx_flat = x.reshape(-1).astype(jnp.int32)
    out = _sc_embed(table, idx_flat)
    return out.reshape(b, s, table.shape[1])
